# baseline (speedup 1.0000x reference)
import functools

import jax
import jax.numpy as jnp
from jax import lax
from jax.experimental import pallas as pl
from jax.experimental.pallas import tpu as pltpu
from jax.experimental.pallas import tpu_sc as plsc

N_USERS = 4000
N_ITEMS = 6000
N_NODES = N_USERS + N_ITEMS
D = 128
NLANE = 16
NDV = D // NLANE

NC = 2
NS = 16
NW = NC * NS

ECHUNK = 80
SUP = 25

_MESH = plsc.VectorSubcoreMesh(core_axis_name="c", subcore_axis_name="s")


def _lane_shuffle(v, idx):
    dnums = lax.GatherDimensionNumbers(
        offset_dims=(), collapsed_slice_dims=(0,), start_index_map=(0,))
    return lax.gather(v, idx[:, None], dnums, (1,),
                      mode=lax.GatherScatterMode.PROMISE_IN_BOUNDS)

RPT = 624
RPT_LAST = N_NODES - (NS - 1) * RPT


def _spmm_body(emb_hbm, src_hbm, dst_hbm, w_hbm, p0_hbm, p1_hbm,
               acc_sh, sidx_v, didx_v, wv_v, rows_v, sem):
    core = lax.axis_index("c")
    sub = lax.axis_index("s")
    wid = sub * NC + core

    n_stages = src_hbm.shape[1]

    def _init_slice(base, n):
        pltpu.sync_copy(emb_hbm.at[pl.ds(base, n)],
                        acc_sh.at[pl.ds(base, n)])

    @pl.when(sub < NS - 1)
    def _():
        _init_slice(pl.multiple_of(sub * RPT, 8), RPT)

    @pl.when(sub == NS - 1)
    def _():
        _init_slice((NS - 1) * RPT, RPT_LAST)

    plsc.subcore_barrier()

    def stage_body(s, carry0):
        pltpu.sync_copy(src_hbm.at[wid, s], sidx_v)
        pltpu.sync_copy(dst_hbm.at[wid, s], didx_v)
        pltpu.sync_copy(w_hbm.at[wid, s], wv_v)

        pltpu.async_copy(emb_hbm.at[sidx_v.at[0]], rows_v.at[0], sem)

        def chunk_body(j, carry):
            b = lax.rem(j, 2)
            pltpu.make_async_copy(emb_hbm.at[sidx_v.at[j]],
                                  rows_v.at[b], sem).wait()

            @pl.when(j < SUP - 1)
            def _():
                pltpu.async_copy(emb_hbm.at[sidx_v.at[j + 1]],
                                 rows_v.at[1 - b], sem)

            def group_body(g, c2):
                wv16 = wv_v[j, pl.ds(g * NLANE, NLANE)]
                for e in range(NLANE):
                    w = wv16[e]
                    row = g * NLANE + e
                    for d in range(NDV):
                        sl = pl.ds(d * NLANE, NLANE)
                        rows_v[b, row, sl] = rows_v[b, row, sl] * w
                return c2
            lax.fori_loop(0, ECHUNK // NLANE, group_body, 0)

            pltpu.sync_copy(rows_v.at[b], acc_sh.at[didx_v.at[j]], add=True)
            return carry

        lax.fori_loop(0, SUP, chunk_body, 0)
        return carry0

    lax.fori_loop(0, n_stages, stage_body, 0)
    plsc.subcore_barrier()

    def _writeback(dst_hbm_ref):
        @pl.when(sub < NS - 1)
        def _():
            base = pl.multiple_of(sub * RPT, 8)
            pltpu.sync_copy(acc_sh.at[pl.ds(base, RPT)],
                            dst_hbm_ref.at[pl.ds(base, RPT)])

        @pl.when(sub == NS - 1)
        def _():
            base = (NS - 1) * RPT
            pltpu.sync_copy(acc_sh.at[pl.ds(base, RPT_LAST)],
                            dst_hbm_ref.at[pl.ds(base, RPT_LAST)])

    @pl.when(core == 0)
    def _():
        _writeback(p0_hbm)

    @pl.when(core == 1)
    def _():
        _writeback(p1_hbm)


def _score_body(p0_hbm, p1_hbm, emb_hbm, un_hbm, in_hbm, gamma_hbm,
                uidx, iidx, gu0, gu1, gue, gi0, gi1, gie, gout, sem):
    core = lax.axis_index("c")
    sub = lax.axis_index("s")
    wid = sub * NC + core
    ppw = uidx.shape[0]

    pltpu.sync_copy(un_hbm.at[wid], uidx)
    pltpu.sync_copy(in_hbm.at[wid], iidx)

    pltpu.async_copy(p0_hbm.at[uidx], gu0, sem).wait()
    pltpu.async_copy(p1_hbm.at[uidx], gu1, sem).wait()
    pltpu.async_copy(emb_hbm.at[uidx], gue, sem).wait()
    pltpu.async_copy(p0_hbm.at[iidx], gi0, sem).wait()
    pltpu.async_copy(p1_hbm.at[iidx], gi1, sem).wait()
    pltpu.async_copy(emb_hbm.at[iidx], gie, sem).wait()

    def group_body(g, carry):
        lanes = lax.iota(jnp.int32, NLANE)
        out16 = None
        for e in range(NLANE):
            p = g * NLANE + e
            vacc = None
            for d in range(NDV):
                sl = pl.ds(d * NLANE, NLANE)
                vu = gu0[p, sl] + gu1[p, sl] - gue[p, sl]
                vi = gi0[p, sl] + gi1[p, sl] - gie[p, sl]
                prod = vu * vi
                vacc = prod if vacc is None else vacc + prod
            for sh in (1, 2, 4, 8):
                vacc = vacc + _lane_shuffle(vacc, lanes ^ sh)
            oh = (1 - jnp.minimum(jnp.abs(lanes - e), 1)).astype(jnp.float32)
            term = vacc * (0.25 * oh)
            out16 = term if out16 is None else out16 + term
        gout[pl.ds(g * NLANE, NLANE)] = out16
        return carry

    lax.fori_loop(0, ppw // NLANE, group_body, 0)
    pltpu.sync_copy(gout, gamma_hbm.at[pl.ds(wid * ppw, ppw)])


def kernel(users, items, user_emb, item_emb, edge_index, edge_weight):
    n_edges = edge_index.shape[1]
    n_pairs = users.shape[0]
    chunks_per_w = n_edges // (NW * ECHUNK)
    ppw = n_pairs // NW

    n_stages = chunks_per_w // SUP
    all_emb = jnp.concatenate([user_emb, item_emb], axis=0)
    src2 = edge_index[1].reshape(NW, n_stages, SUP, ECHUNK)
    dst2 = edge_index[0].reshape(NW, n_stages, SUP, ECHUNK)
    w2 = edge_weight.reshape(NW, n_stages, SUP, ECHUNK)
    un2 = users.reshape(NW, ppw)
    in2 = (items + N_USERS).reshape(NW, ppw)

    spmm = pl.kernel(
        _spmm_body,
        out_type=[jax.ShapeDtypeStruct((N_NODES, D), jnp.float32),
                  jax.ShapeDtypeStruct((N_NODES, D), jnp.float32)],
        mesh=_MESH,
        scratch_types=[
            pltpu.VMEM_SHARED((N_NODES, D), jnp.float32),
            pltpu.VMEM((SUP, ECHUNK), jnp.int32),
            pltpu.VMEM((SUP, ECHUNK), jnp.int32),
            pltpu.VMEM((SUP, ECHUNK), jnp.float32),
            pltpu.VMEM((2, ECHUNK, D), jnp.float32),
            pltpu.SemaphoreType.DMA,
        ],
    )
    p0, p1 = spmm(all_emb, src2, dst2, w2)

    score = pl.kernel(
        _score_body,
        out_type=jax.ShapeDtypeStruct((n_pairs,), jnp.float32),
        mesh=_MESH,
        scratch_types=[
            pltpu.VMEM((ppw,), jnp.int32),
            pltpu.VMEM((ppw,), jnp.int32),
            pltpu.VMEM((ppw, D), jnp.float32),
            pltpu.VMEM((ppw, D), jnp.float32),
            pltpu.VMEM((ppw, D), jnp.float32),
            pltpu.VMEM((ppw, D), jnp.float32),
            pltpu.VMEM((ppw, D), jnp.float32),
            pltpu.VMEM((ppw, D), jnp.float32),
            pltpu.VMEM((ppw,), jnp.float32),
            pltpu.SemaphoreType.DMA,
        ],
    )
    return score(p0, p1, all_emb, un2, in2)


# speedup vs baseline: 2.3942x; 2.3942x over previous
import functools

import jax
import jax.numpy as jnp
from jax import lax
from jax.experimental import pallas as pl
from jax.experimental.pallas import tpu as pltpu
from jax.experimental.pallas import tpu_sc as plsc

N_USERS = 4000
N_ITEMS = 6000
N_NODES = N_USERS + N_ITEMS
D = 128
NLANE = 16
NDV = D // NLANE

NC = 2
NS = 16
NW = NC * NS

ECHUNK = 128
CPW = 80
SUP = 20
E_PAD = NW * CPW * ECHUNK

_MESH = plsc.VectorSubcoreMesh(core_axis_name="c", subcore_axis_name="s")


def _lane_shuffle(v, idx):
    dnums = lax.GatherDimensionNumbers(
        offset_dims=(), collapsed_slice_dims=(0,), start_index_map=(0,))
    return lax.gather(v, idx[:, None], dnums, (1,),
                      mode=lax.GatherScatterMode.PROMISE_IN_BOUNDS)

RPT = 624
RPT_LAST = N_NODES - (NS - 1) * RPT


def _scale_chunk(src_ref, dst_ref, wv_v, j):
    def group_body(g, c2):
        wv16 = wv_v[j, pl.ds(g * NLANE, NLANE)]
        for e in range(NLANE):
            w = wv16[e]
            row = g * NLANE + e
            for d in range(NDV):
                sl = pl.ds(d * NLANE, NLANE)
                dst_ref[row, sl] = src_ref[row, sl] * w
        return c2
    lax.fori_loop(0, ECHUNK // NLANE, group_body, 0, unroll=4)


def _spmm_body(emb_hbm, src_hbm, dst_hbm, w_hbm, p0_hbm, p1_hbm,
               acc_sh, sidx_v, didx_v, wv_v, rows0_v, rows1_v,
               g0sem, g1sem, s0sem, s1sem):
    core = lax.axis_index("c")
    sub = lax.axis_index("s")
    wid = sub * NC + core

    n_stages = src_hbm.shape[1]

    def _init_slice(base, n):
        pltpu.sync_copy(emb_hbm.at[pl.ds(base, n)],
                        acc_sh.at[pl.ds(base, n)])

    @pl.when(sub < NS - 1)
    def _():
        _init_slice(pl.multiple_of(sub * RPT, 8), RPT)

    @pl.when(sub == NS - 1)
    def _():
        _init_slice((NS - 1) * RPT, RPT_LAST)

    plsc.subcore_barrier()

    def stage_body(s, carry0):
        pltpu.sync_copy(src_hbm.at[wid, s], sidx_v)
        pltpu.sync_copy(dst_hbm.at[wid, s], didx_v)
        pltpu.sync_copy(w_hbm.at[wid, s], wv_v)

        pltpu.async_copy(emb_hbm.at[sidx_v.at[0]], rows0_v, g0sem)

        def pair_body(k, carry):
            j0 = 2 * k
            j1 = j0 + 1

            pltpu.async_copy(emb_hbm.at[sidx_v.at[j1]], rows1_v, g1sem)
            pltpu.make_async_copy(emb_hbm.at[sidx_v.at[j0]],
                                  rows0_v, g0sem).wait()
            _scale_chunk(rows0_v, rows0_v, wv_v, j0)
            pltpu.sync_copy(rows0_v, acc_sh.at[didx_v.at[j0]], add=True)

            @pl.when(k < SUP // 2 - 1)
            def _():
                pltpu.async_copy(emb_hbm.at[sidx_v.at[j0 + 2]],
                                 rows0_v, g0sem)
            pltpu.make_async_copy(emb_hbm.at[sidx_v.at[j1]],
                                  rows1_v, g1sem).wait()
            _scale_chunk(rows1_v, rows1_v, wv_v, j1)
            pltpu.sync_copy(rows1_v, acc_sh.at[didx_v.at[j1]], add=True)
            return carry

        lax.fori_loop(0, SUP // 2, pair_body, 0)
        return carry0

    lax.fori_loop(0, n_stages, stage_body, 0)
    plsc.subcore_barrier()

    def _writeback(dst_hbm_ref):
        @pl.when(sub < NS - 1)
        def _():
            base = pl.multiple_of(sub * RPT, 8)
            pltpu.sync_copy(acc_sh.at[pl.ds(base, RPT)],
                            dst_hbm_ref.at[pl.ds(base, RPT)])

        @pl.when(sub == NS - 1)
        def _():
            base = (NS - 1) * RPT
            pltpu.sync_copy(acc_sh.at[pl.ds(base, RPT_LAST)],
                            dst_hbm_ref.at[pl.ds(base, RPT_LAST)])

    @pl.when(core == 0)
    def _():
        _writeback(p0_hbm)

    @pl.when(core == 1)
    def _():
        _writeback(p1_hbm)


def _score_body(p0_hbm, p1_hbm, emb_hbm, un_hbm, in_hbm, gamma_hbm,
                uidx, iidx, gu0, gu1, gue, gi0, gi1, gie, gout, sem):
    core = lax.axis_index("c")
    sub = lax.axis_index("s")
    wid = sub * NC + core
    ppw = uidx.shape[0]

    pltpu.sync_copy(un_hbm.at[wid], uidx)
    pltpu.sync_copy(in_hbm.at[wid], iidx)

    pltpu.async_copy(p0_hbm.at[uidx], gu0, sem)
    pltpu.async_copy(p1_hbm.at[uidx], gu1, sem)
    pltpu.async_copy(emb_hbm.at[uidx], gue, sem)
    pltpu.async_copy(p0_hbm.at[iidx], gi0, sem)
    pltpu.async_copy(p1_hbm.at[iidx], gi1, sem)
    pltpu.async_copy(emb_hbm.at[iidx], gie, sem)
    pltpu.make_async_copy(p0_hbm.at[uidx], gu0, sem).wait()
    pltpu.make_async_copy(p1_hbm.at[uidx], gu1, sem).wait()
    pltpu.make_async_copy(emb_hbm.at[uidx], gue, sem).wait()
    pltpu.make_async_copy(p0_hbm.at[iidx], gi0, sem).wait()
    pltpu.make_async_copy(p1_hbm.at[iidx], gi1, sem).wait()
    pltpu.make_async_copy(emb_hbm.at[iidx], gie, sem).wait()

    def group_body(g, carry):
        lanes = lax.iota(jnp.int32, NLANE)
        out16 = None
        for e in range(NLANE):
            p = g * NLANE + e
            vacc = None
            for d in range(NDV):
                sl = pl.ds(d * NLANE, NLANE)
                vu = gu0[p, sl] + gu1[p, sl] - gue[p, sl]
                vi = gi0[p, sl] + gi1[p, sl] - gie[p, sl]
                prod = vu * vi
                vacc = prod if vacc is None else vacc + prod
            for sh in (1, 2, 4, 8):
                vacc = vacc + _lane_shuffle(vacc, lanes ^ sh)
            oh = (1 - jnp.minimum(jnp.abs(lanes - e), 1)).astype(jnp.float32)
            term = vacc * (0.25 * oh)
            out16 = term if out16 is None else out16 + term
        gout[pl.ds(g * NLANE, NLANE)] = out16
        return carry

    lax.fori_loop(0, ppw // NLANE, group_body, 0)
    pltpu.sync_copy(gout, gamma_hbm.at[pl.ds(wid * ppw, ppw)])


def kernel(users, items, user_emb, item_emb, edge_index, edge_weight):
    n_edges = edge_index.shape[1]
    n_pairs = users.shape[0]
    ppw = n_pairs // NW

    n_stages = CPW // SUP
    pad = E_PAD - n_edges
    all_emb = jnp.concatenate([user_emb, item_emb], axis=0)
    fill = (jnp.arange(pad, dtype=jnp.int32) * 16) % N_NODES
    src_p = jnp.concatenate([edge_index[1], fill])
    dst_p = jnp.concatenate([edge_index[0], fill])
    w_p = jnp.pad(edge_weight, (0, pad))
    src2 = src_p.reshape(NW, n_stages, SUP, ECHUNK)
    dst2 = dst_p.reshape(NW, n_stages, SUP, ECHUNK)
    w2 = w_p.reshape(NW, n_stages, SUP, ECHUNK)
    un2 = users.reshape(NW, ppw)
    in2 = (items + N_USERS).reshape(NW, ppw)

    spmm = pl.kernel(
        _spmm_body,
        out_type=[jax.ShapeDtypeStruct((N_NODES, D), jnp.float32),
                  jax.ShapeDtypeStruct((N_NODES, D), jnp.float32)],
        mesh=_MESH,
        scratch_types=[
            pltpu.VMEM_SHARED((N_NODES, D), jnp.float32),
            pltpu.VMEM((SUP, ECHUNK), jnp.int32),
            pltpu.VMEM((SUP, ECHUNK), jnp.int32),
            pltpu.VMEM((SUP, ECHUNK), jnp.float32),
            pltpu.VMEM((ECHUNK, D), jnp.float32),
            pltpu.VMEM((ECHUNK, D), jnp.float32),
            pltpu.SemaphoreType.DMA,
            pltpu.SemaphoreType.DMA,
            pltpu.SemaphoreType.DMA,
            pltpu.SemaphoreType.DMA,
        ],
    )
    p0, p1 = spmm(all_emb, src2, dst2, w2)

    score = pl.kernel(
        _score_body,
        out_type=jax.ShapeDtypeStruct((n_pairs,), jnp.float32),
        mesh=_MESH,
        scratch_types=[
            pltpu.VMEM((ppw,), jnp.int32),
            pltpu.VMEM((ppw,), jnp.int32),
            pltpu.VMEM((ppw, D), jnp.float32),
            pltpu.VMEM((ppw, D), jnp.float32),
            pltpu.VMEM((ppw, D), jnp.float32),
            pltpu.VMEM((ppw, D), jnp.float32),
            pltpu.VMEM((ppw, D), jnp.float32),
            pltpu.VMEM((ppw, D), jnp.float32),
            pltpu.VMEM((ppw,), jnp.float32),
            pltpu.SemaphoreType.DMA,
        ],
    )
    return score(p0, p1, all_emb, un2, in2)


# speedup vs baseline: 2.5620x; 1.0701x over previous
import functools

import jax
import jax.numpy as jnp
from jax import lax
from jax.experimental import pallas as pl
from jax.experimental.pallas import tpu as pltpu
from jax.experimental.pallas import tpu_sc as plsc

N_USERS = 4000
N_ITEMS = 6000
N_NODES = N_USERS + N_ITEMS
D = 128
NLANE = 16
NDV = D // NLANE

NC = 2
NS = 16
NW = NC * NS

ECHUNK = 128
CPW = 80
SUP = 40
E_PAD = NW * CPW * ECHUNK

_MESH = plsc.VectorSubcoreMesh(core_axis_name="c", subcore_axis_name="s")


def _lane_shuffle(v, idx):
    dnums = lax.GatherDimensionNumbers(
        offset_dims=(), collapsed_slice_dims=(0,), start_index_map=(0,))
    return lax.gather(v, idx[:, None], dnums, (1,),
                      mode=lax.GatherScatterMode.PROMISE_IN_BOUNDS)

RPT = 624
RPT_LAST = N_NODES - (NS - 1) * RPT


def _scale_chunk(src_ref, dst_ref, wv_v, j):
    def group_body(g, c2):
        wv16 = wv_v[j, pl.ds(g * NLANE, NLANE)]
        for e in range(NLANE):
            w = wv16[e]
            row = g * NLANE + e
            for d in range(NDV):
                sl = pl.ds(d * NLANE, NLANE)
                dst_ref[row, sl] = src_ref[row, sl] * w
        return c2
    lax.fori_loop(0, ECHUNK // NLANE, group_body, 0)


def _spmm_body(emb_hbm, src_hbm, dst_hbm, w_hbm, p0_hbm, p1_hbm,
               acc_sh, sidx_v, didx_v, wv_v, rows0_v, rows1_v,
               g0sem, g1sem, s0sem, s1sem):
    core = lax.axis_index("c")
    sub = lax.axis_index("s")
    wid = sub * NC + core

    n_stages = src_hbm.shape[1]

    def _init_slice(base, n):
        pltpu.sync_copy(emb_hbm.at[pl.ds(base, n)],
                        acc_sh.at[pl.ds(base, n)])

    @pl.when(sub < NS - 1)
    def _():
        _init_slice(pl.multiple_of(sub * RPT, 8), RPT)

    @pl.when(sub == NS - 1)
    def _():
        _init_slice((NS - 1) * RPT, RPT_LAST)

    plsc.subcore_barrier()

    def stage_body(s, carry0):
        pltpu.sync_copy(src_hbm.at[wid, s], sidx_v)
        pltpu.sync_copy(dst_hbm.at[wid, s], didx_v)
        pltpu.sync_copy(w_hbm.at[wid, s], wv_v)

        pltpu.async_copy(emb_hbm.at[sidx_v.at[0]], rows0_v, g0sem)

        def pair_body(k, carry):
            j0 = 2 * k
            j1 = j0 + 1

            pltpu.async_copy(emb_hbm.at[sidx_v.at[j1]], rows1_v, g1sem)
            pltpu.make_async_copy(emb_hbm.at[sidx_v.at[j0]],
                                  rows0_v, g0sem).wait()
            _scale_chunk(rows0_v, rows0_v, wv_v, j0)
            pltpu.sync_copy(rows0_v, acc_sh.at[didx_v.at[j0]], add=True)

            @pl.when(k < SUP // 2 - 1)
            def _():
                pltpu.async_copy(emb_hbm.at[sidx_v.at[j0 + 2]],
                                 rows0_v, g0sem)
            pltpu.make_async_copy(emb_hbm.at[sidx_v.at[j1]],
                                  rows1_v, g1sem).wait()
            _scale_chunk(rows1_v, rows1_v, wv_v, j1)
            pltpu.sync_copy(rows1_v, acc_sh.at[didx_v.at[j1]], add=True)
            return carry

        lax.fori_loop(0, SUP // 2, pair_body, 0)
        return carry0

    lax.fori_loop(0, n_stages, stage_body, 0)
    plsc.subcore_barrier()

    def _writeback(dst_hbm_ref):
        @pl.when(sub < NS - 1)
        def _():
            base = pl.multiple_of(sub * RPT, 8)
            pltpu.sync_copy(acc_sh.at[pl.ds(base, RPT)],
                            dst_hbm_ref.at[pl.ds(base, RPT)])

        @pl.when(sub == NS - 1)
        def _():
            base = (NS - 1) * RPT
            pltpu.sync_copy(acc_sh.at[pl.ds(base, RPT_LAST)],
                            dst_hbm_ref.at[pl.ds(base, RPT_LAST)])

    @pl.when(core == 0)
    def _():
        _writeback(p0_hbm)

    @pl.when(core == 1)
    def _():
        _writeback(p1_hbm)


def _score_body(p0_hbm, p1_hbm, emb_hbm, un_hbm, in_hbm, gamma_hbm,
                uidx, iidx, gu0, gu1, gue, gi0, gi1, gie, gout, sem):
    core = lax.axis_index("c")
    sub = lax.axis_index("s")
    wid = sub * NC + core
    ppw = uidx.shape[0]

    pltpu.sync_copy(un_hbm.at[wid], uidx)
    pltpu.sync_copy(in_hbm.at[wid], iidx)

    pltpu.async_copy(p0_hbm.at[uidx], gu0, sem)
    pltpu.async_copy(p1_hbm.at[uidx], gu1, sem)
    pltpu.async_copy(emb_hbm.at[uidx], gue, sem)
    pltpu.async_copy(p0_hbm.at[iidx], gi0, sem)
    pltpu.async_copy(p1_hbm.at[iidx], gi1, sem)
    pltpu.async_copy(emb_hbm.at[iidx], gie, sem)
    pltpu.make_async_copy(p0_hbm.at[uidx], gu0, sem).wait()
    pltpu.make_async_copy(p1_hbm.at[uidx], gu1, sem).wait()
    pltpu.make_async_copy(emb_hbm.at[uidx], gue, sem).wait()
    pltpu.make_async_copy(p0_hbm.at[iidx], gi0, sem).wait()
    pltpu.make_async_copy(p1_hbm.at[iidx], gi1, sem).wait()
    pltpu.make_async_copy(emb_hbm.at[iidx], gie, sem).wait()

    def group_body(g, carry):
        lanes = lax.iota(jnp.int32, NLANE)
        out16 = None
        for e in range(NLANE):
            p = g * NLANE + e
            vacc = None
            for d in range(NDV):
                sl = pl.ds(d * NLANE, NLANE)
                vu = gu0[p, sl] + gu1[p, sl] - gue[p, sl]
                vi = gi0[p, sl] + gi1[p, sl] - gie[p, sl]
                prod = vu * vi
                vacc = prod if vacc is None else vacc + prod
            for sh in (1, 2, 4, 8):
                vacc = vacc + _lane_shuffle(vacc, lanes ^ sh)
            oh = (1 - jnp.minimum(jnp.abs(lanes - e), 1)).astype(jnp.float32)
            term = vacc * (0.25 * oh)
            out16 = term if out16 is None else out16 + term
        gout[pl.ds(g * NLANE, NLANE)] = out16
        return carry

    lax.fori_loop(0, ppw // NLANE, group_body, 0)
    pltpu.sync_copy(gout, gamma_hbm.at[pl.ds(wid * ppw, ppw)])


def kernel(users, items, user_emb, item_emb, edge_index, edge_weight):
    n_edges = edge_index.shape[1]
    n_pairs = users.shape[0]
    ppw = n_pairs // NW

    n_stages = CPW // SUP
    pad = E_PAD - n_edges
    all_emb = jnp.concatenate([user_emb, item_emb], axis=0)
    fill = (jnp.arange(pad, dtype=jnp.int32) * 16) % N_NODES
    src_p = jnp.concatenate([edge_index[1], fill])
    dst_p = jnp.concatenate([edge_index[0], fill])
    w_p = jnp.pad(edge_weight, (0, pad))
    src2 = src_p.reshape(NW, n_stages, SUP, ECHUNK)
    dst2 = dst_p.reshape(NW, n_stages, SUP, ECHUNK)
    w2 = w_p.reshape(NW, n_stages, SUP, ECHUNK)
    un2 = users.reshape(NW, ppw)
    in2 = (items + N_USERS).reshape(NW, ppw)

    spmm = pl.kernel(
        _spmm_body,
        out_type=[jax.ShapeDtypeStruct((N_NODES, D), jnp.float32),
                  jax.ShapeDtypeStruct((N_NODES, D), jnp.float32)],
        mesh=_MESH,
        scratch_types=[
            pltpu.VMEM_SHARED((N_NODES, D), jnp.float32),
            pltpu.VMEM((SUP, ECHUNK), jnp.int32),
            pltpu.VMEM((SUP, ECHUNK), jnp.int32),
            pltpu.VMEM((SUP, ECHUNK), jnp.float32),
            pltpu.VMEM((ECHUNK, D), jnp.float32),
            pltpu.VMEM((ECHUNK, D), jnp.float32),
            pltpu.SemaphoreType.DMA,
            pltpu.SemaphoreType.DMA,
            pltpu.SemaphoreType.DMA,
            pltpu.SemaphoreType.DMA,
        ],
    )
    p0, p1 = spmm(all_emb, src2, dst2, w2)

    score = pl.kernel(
        _score_body,
        out_type=jax.ShapeDtypeStruct((n_pairs,), jnp.float32),
        mesh=_MESH,
        scratch_types=[
            pltpu.VMEM((ppw,), jnp.int32),
            pltpu.VMEM((ppw,), jnp.int32),
            pltpu.VMEM((ppw, D), jnp.float32),
            pltpu.VMEM((ppw, D), jnp.float32),
            pltpu.VMEM((ppw, D), jnp.float32),
            pltpu.VMEM((ppw, D), jnp.float32),
            pltpu.VMEM((ppw, D), jnp.float32),
            pltpu.VMEM((ppw, D), jnp.float32),
            pltpu.VMEM((ppw,), jnp.float32),
            pltpu.SemaphoreType.DMA,
        ],
    )
    return score(p0, p1, all_emb, un2, in2)
